# baseline (speedup 1.0000x reference)
import functools

import jax
import jax.numpy as jnp
from jax import lax
from jax.experimental import pallas as pl
from jax.experimental.pallas import tpu as pltpu
from jax.experimental.pallas import tpu_sc as plsc

VOCAB = 100000
EMB = 128
DFEAT = 512
NCLS = 64
B = 4096
BLK = 512

_NC = 2
_NS = 16
_NW = _NC * _NS
_BPW = B // _NW

_VMEM = pltpu.MemorySpace.VMEM


def _sc_gather(table, idx):
    mesh = plsc.VectorSubcoreMesh(core_axis_name="c", subcore_axis_name="s")

    @functools.partial(
        pl.kernel,
        out_type=jax.ShapeDtypeStruct((B, EMB), jnp.float32),
        mesh=mesh,
        scratch_types=[
            pltpu.VMEM((_BPW,), jnp.int32),
            pltpu.VMEM((_BPW, EMB), jnp.float32),
            pltpu.SemaphoreType.DMA,
        ],
    )
    def gather_kernel(table_hbm, idx_hbm, out_hbm, idx_v, rows_v, sem):
        wid = lax.axis_index("s") * _NC + lax.axis_index("c")
        base = wid * _BPW
        pltpu.sync_copy(idx_hbm.at[pl.ds(base, _BPW)], idx_v)
        pltpu.async_copy(table_hbm.at[idx_v], rows_v, sem).wait()
        pltpu.sync_copy(rows_v, out_hbm.at[pl.ds(base, _BPW)])

    return gather_kernel(table, idx)


def _dot_t(x, wt):
    return lax.dot_general(x, wt, (((1,), (1,)), ((), ())),
                           preferred_element_type=jnp.float32)


def _feat_body(f0_ref, f1_ref, wkt_ref, bk_ref, wpt_ref, o_ref):
    wkt = wkt_ref[...]
    wpt = wpt_ref[:, :NCLS]
    bk = bk_ref[...]
    half = B // 2
    for i, f_ref in enumerate((f0_ref, f1_ref)):
        h_f = jnp.maximum(_dot_t(f_ref[...], wkt) + bk, 0.0)
        o_ref[pl.ds(i * half, half), :] = _dot_t(h_f, wpt)


def _tc_feat(features, WkT, bk, WpT):
    half = B // 2
    return pl.pallas_call(
        _feat_body,
        in_specs=[pl.BlockSpec(memory_space=_VMEM)] * 5,
        out_specs=pl.BlockSpec(memory_space=_VMEM),
        out_shape=jax.ShapeDtypeStruct((B, NCLS), jnp.float32),
    )(features[:half], features[half:], WkT, bk, WpT)


def _combine_body(a_ref, e_ref, wlt_ref, bl_ref, wpt_ref, bp_ref, o_ref):
    wlt = wlt_ref[...]
    wpt = wpt_ref[:, NCLS:]
    bl = bl_ref[...]
    bp = bp_ref[...]
    cblk = 4096
    for i in range(B // cblk):
        e_blk = e_ref[pl.ds(i * cblk, cblk), :]
        h_e = jnp.maximum(_dot_t(e_blk, wlt) + bl, 0.0)
        logits = a_ref[pl.ds(i * cblk, cblk), :] + _dot_t(h_e, wpt) + bp
        m = jnp.max(logits, axis=-1, keepdims=True)
        p = jnp.exp(logits - m)
        p = p / jnp.sum(p, axis=-1, keepdims=True)
        o_ref[:, pl.ds(i * cblk, cblk)] = jnp.transpose(p, (1, 0))


def _tc_combine(a_f, embs, WlT, bl, WpT, bp):
    return pl.pallas_call(
        _combine_body,
        in_specs=[pl.BlockSpec(memory_space=_VMEM)] * 6,
        out_specs=pl.BlockSpec(memory_space=_VMEM),
        out_shape=jax.ShapeDtypeStruct((NCLS, B), jnp.float32),
    )(a_f, embs, WlT, bl, WpT, bp)


def kernel(features, indices, table, Wk, bk, Wl, bl, Wp, bp):
    embs = _sc_gather(table, indices.astype(jnp.int32))
    a_f = _tc_feat(features, Wk.T, bk, Wp.T)
    out_t = _tc_combine(a_f, embs, Wl.T, bl, Wp.T, bp)
    return out_t.T


# speedup vs baseline: 1.3166x; 1.3166x over previous
import functools

import jax
import jax.numpy as jnp
from jax import lax
from jax.experimental import pallas as pl
from jax.experimental.pallas import tpu as pltpu
from jax.experimental.pallas import tpu_sc as plsc

VOCAB = 100000
EMB = 128
DFEAT = 512
NCLS = 64
B = 4096
BLK = 512

_NC = 2
_NS = 16
_NW = _NC * _NS
_BPW = B // _NW

_VMEM = pltpu.MemorySpace.VMEM


def _sc_gather(table, idx):
    mesh = plsc.VectorSubcoreMesh(core_axis_name="c", subcore_axis_name="s")

    @functools.partial(
        pl.kernel,
        out_type=jax.ShapeDtypeStruct((B, EMB), jnp.float32),
        mesh=mesh,
        scratch_types=[
            pltpu.VMEM((_BPW,), jnp.int32),
            pltpu.VMEM((_BPW, EMB), jnp.float32),
            pltpu.SemaphoreType.DMA,
        ],
    )
    def gather_kernel(table_hbm, idx_hbm, out_hbm, idx_v, rows_v, sem):
        wid = lax.axis_index("s") * _NC + lax.axis_index("c")
        base = wid * _BPW
        pltpu.sync_copy(idx_hbm.at[pl.ds(base, _BPW)], idx_v)
        pltpu.async_copy(table_hbm.at[idx_v], rows_v, sem).wait()
        pltpu.sync_copy(rows_v, out_hbm.at[pl.ds(base, _BPW)])

    return gather_kernel(table, idx)


def _dot_t(x, wt):
    return lax.dot_general(x, wt, (((1,), (1,)), ((), ())),
                           preferred_element_type=jnp.float32)


def _feat_body(f_ref, wkt_ref, bk_ref, wpt_ref, o_ref):
    wkt = wkt_ref[...]
    wpt = wpt_ref[:, :NCLS]
    bk = bk_ref[...]
    h_f = jnp.maximum(_dot_t(f_ref[...], wkt) + bk, 0.0)
    o_ref[...] = _dot_t(h_f, wpt)


def _tc_feat(features, WkT, bk, WpT):
    return pl.pallas_call(
        _feat_body,
        in_specs=[pl.BlockSpec(memory_space=_VMEM)] * 4,
        out_specs=pl.BlockSpec(memory_space=_VMEM),
        out_shape=jax.ShapeDtypeStruct((B, NCLS), jnp.float32),
    )(features, WkT, bk, WpT)


def _combine_body(a_ref, e_ref, wlt_ref, bl_ref, wpt_ref, bp_ref, o_ref):
    wlt = wlt_ref[...]
    wpt = wpt_ref[:, NCLS:]
    bl = bl_ref[...]
    bp = bp_ref[...]
    cblk = 4096
    for i in range(B // cblk):
        e_blk = e_ref[pl.ds(i * cblk, cblk), :]
        h_e = jnp.maximum(_dot_t(e_blk, wlt) + bl, 0.0)
        logits = a_ref[pl.ds(i * cblk, cblk), :] + _dot_t(h_e, wpt) + bp
        m = jnp.max(logits, axis=-1, keepdims=True)
        p = jnp.exp(logits - m)
        p = p / jnp.sum(p, axis=-1, keepdims=True)
        o_ref[:, pl.ds(i * cblk, cblk)] = jnp.transpose(p, (1, 0))


def _tc_combine(a_f, embs, WlT, bl, WpT, bp):
    return pl.pallas_call(
        _combine_body,
        in_specs=[pl.BlockSpec(memory_space=_VMEM)] * 6,
        out_specs=pl.BlockSpec(memory_space=_VMEM),
        out_shape=jax.ShapeDtypeStruct((NCLS, B), jnp.float32),
    )(a_f, embs, WlT, bl, WpT, bp)


def kernel(features, indices, table, Wk, bk, Wl, bl, Wp, bp):
    embs = _sc_gather(table, indices.astype(jnp.int32))
    a_f = _tc_feat(features, Wk.T, bk, Wp.T)
    out_t = _tc_combine(a_f, embs, Wl.T, bl, Wp.T, bp)
    return out_t.T


# speedup vs baseline: 1.3751x; 1.0444x over previous
import functools

import jax
import jax.numpy as jnp
from jax import lax
from jax.experimental import pallas as pl
from jax.experimental.pallas import tpu as pltpu
from jax.experimental.pallas import tpu_sc as plsc

VOCAB = 100000
EMB = 128
DFEAT = 512
NCLS = 64
B = 4096
BLK = 512

_NC = 2
_NS = 16
_NW = _NC * _NS
_BPW = B // _NW

_VMEM = pltpu.MemorySpace.VMEM


def _sc_gather(table, idx):
    mesh = plsc.VectorSubcoreMesh(core_axis_name="c", subcore_axis_name="s")

    @functools.partial(
        pl.kernel,
        out_type=jax.ShapeDtypeStruct((B, EMB), jnp.float32),
        mesh=mesh,
        scratch_types=[
            pltpu.VMEM((_BPW,), jnp.int32),
            pltpu.VMEM((_BPW, EMB), jnp.float32),
            pltpu.SemaphoreType.DMA,
        ],
    )
    def gather_kernel(table_hbm, idx_hbm, out_hbm, idx_v, rows_v, sem):
        wid = lax.axis_index("s") * _NC + lax.axis_index("c")
        base = wid * _BPW
        pltpu.sync_copy(idx_hbm.at[pl.ds(base, _BPW)], idx_v)
        pltpu.async_copy(table_hbm.at[idx_v], rows_v, sem).wait()
        pltpu.sync_copy(rows_v, out_hbm.at[pl.ds(base, _BPW)])

    return gather_kernel(table, idx)


def _dot_t(x, wt):
    return lax.dot_general(x, wt, (((1,), (1,)), ((), ())),
                           preferred_element_type=jnp.float32)


def _feat_body(f_ref, wkt_ref, bk_ref, wpt_ref, o_ref):
    wkt = wkt_ref[...]
    wpt = wpt_ref[:, :NCLS]
    bk = bk_ref[...]
    h_f = jnp.maximum(_dot_t(f_ref[...], wkt) + bk, 0.0)
    o_ref[...] = _dot_t(wpt, h_f)


def _tc_feat(features, WkT, bk, WpT):
    return pl.pallas_call(
        _feat_body,
        in_specs=[pl.BlockSpec(memory_space=_VMEM)] * 4,
        out_specs=pl.BlockSpec(memory_space=_VMEM),
        out_shape=jax.ShapeDtypeStruct((NCLS, B), jnp.float32),
    )(features, WkT, bk, WpT)


def _combine_body(at_ref, e_ref, wlt_ref, bl_ref, wpt_ref, bpc_ref, o_ref):
    wlt = wlt_ref[...]
    wpt = wpt_ref[:, NCLS:]
    bl = bl_ref[...]
    bpc = bpc_ref[...]
    h_e = jnp.maximum(_dot_t(e_ref[...], wlt) + bl, 0.0)
    logits_t = at_ref[...] + _dot_t(wpt, h_e) + bpc
    m = jnp.max(logits_t, axis=0, keepdims=True)
    p = jnp.exp(logits_t - m)
    o_ref[...] = p / jnp.sum(p, axis=0, keepdims=True)


def _tc_combine(a_ft, embs, WlT, bl, WpT, bp_col):
    return pl.pallas_call(
        _combine_body,
        in_specs=[pl.BlockSpec(memory_space=_VMEM)] * 6,
        out_specs=pl.BlockSpec(memory_space=_VMEM),
        out_shape=jax.ShapeDtypeStruct((NCLS, B), jnp.float32),
    )(a_ft, embs, WlT, bl, WpT, bp_col)


def kernel(features, indices, table, Wk, bk, Wl, bl, Wp, bp):
    embs = _sc_gather(table, indices.astype(jnp.int32))
    a_ft = _tc_feat(features, Wk.T, bk, Wp.T)
    out_t = _tc_combine(a_ft, embs, Wl.T, bl, Wp.T, bp.reshape(NCLS, 1))
    return out_t.T


# speedup vs baseline: 1.3964x; 1.0155x over previous
import functools

import jax
import jax.numpy as jnp
from jax import lax
from jax.experimental import pallas as pl
from jax.experimental.pallas import tpu as pltpu
from jax.experimental.pallas import tpu_sc as plsc

VOCAB = 100000
EMB = 128
DFEAT = 512
NCLS = 64
B = 4096
BLK = 512

_NC = 2
_NS = 16
_NW = _NC * _NS
_BPW = B // _NW

_VMEM = pltpu.MemorySpace.VMEM


def _sc_gather(table, idx):
    mesh = plsc.VectorSubcoreMesh(core_axis_name="c", subcore_axis_name="s")

    @functools.partial(
        pl.kernel,
        out_type=jax.ShapeDtypeStruct((B, EMB), jnp.float32),
        mesh=mesh,
        scratch_types=[
            pltpu.VMEM((_BPW,), jnp.int32),
            pltpu.VMEM((_BPW, EMB), jnp.float32),
            pltpu.SemaphoreType.DMA,
        ],
    )
    def gather_kernel(table_hbm, idx_hbm, out_hbm, idx_v, rows_v, sem):
        wid = lax.axis_index("s") * _NC + lax.axis_index("c")
        base = wid * _BPW
        pltpu.sync_copy(idx_hbm.at[pl.ds(base, _BPW)], idx_v)
        pltpu.async_copy(table_hbm.at[idx_v], rows_v, sem).wait()
        pltpu.sync_copy(rows_v, out_hbm.at[pl.ds(base, _BPW)])

    return gather_kernel(table, idx)


def _dot_t(x, wt):
    return lax.dot_general(x, wt, (((1,), (1,)), ((), ())),
                           preferred_element_type=jnp.float32)


def _feat_body(f_ref, wkt_ref, bk_ref, wpt_ref, o_ref):
    wkt = wkt_ref[...]
    wpt = wpt_ref[:, :NCLS]
    bk = bk_ref[...]
    h_f = jnp.maximum(_dot_t(f_ref[...], wkt) + bk, 0.0)
    o_ref[...] = _dot_t(wpt, h_f)


def _tc_feat(features, WkT, bk, WpT):
    return pl.pallas_call(
        _feat_body,
        in_specs=[pl.BlockSpec(memory_space=_VMEM)] * 4,
        out_specs=pl.BlockSpec(memory_space=_VMEM),
        out_shape=jax.ShapeDtypeStruct((NCLS, B), jnp.float32),
    )(features, WkT, bk, WpT)


def _combine_body(at_ref, e_ref, wlt_ref, bl_ref, wpt_ref, bpc_ref, o_ref):
    wlt = wlt_ref[...]
    wpt = wpt_ref[:, NCLS:]
    bl = bl_ref[...]
    bpc = jnp.transpose(jnp.reshape(bpc_ref[...], (1, NCLS)), (1, 0))
    h_e = jnp.maximum(_dot_t(e_ref[...], wlt) + bl, 0.0)
    logits_t = at_ref[...] + _dot_t(wpt, h_e) + bpc
    m = jnp.max(logits_t, axis=0, keepdims=True)
    p = jnp.exp(logits_t - m)
    o_ref[...] = p / jnp.sum(p, axis=0, keepdims=True)


def _tc_combine(a_ft, embs, WlT, bl, WpT, bp_col):
    return pl.pallas_call(
        _combine_body,
        in_specs=[pl.BlockSpec(memory_space=_VMEM)] * 6,
        out_specs=pl.BlockSpec(memory_space=_VMEM),
        out_shape=jax.ShapeDtypeStruct((NCLS, B), jnp.float32),
    )(a_ft, embs, WlT, bl, WpT, bp_col)


def kernel(features, indices, table, Wk, bk, Wl, bl, Wp, bp):
    embs = _sc_gather(table, indices.astype(jnp.int32))
    a_ft = _tc_feat(features, Wk.T, bk, Wp.T)
    out_t = _tc_combine(a_ft, embs, Wl.T, bl, Wp.T, bp)
    return out_t.T
